# Initial kernel scaffold; baseline (speedup 1.0000x reference)
import jax
import jax.numpy as jnp
from jax.experimental import pallas as pl


def kernel(z, edge_index, batch, edge_weight, z_table, W1, b1, W2, b2, W3, b3, l1W, l1b, l2W, l2b):
    raise NotImplementedError("write your pallas kernel here")


# speedup vs baseline: 1.7435x; 1.7435x over previous
import jax
import jax.numpy as jnp
from jax import lax
from jax.experimental import pallas as pl
from jax.experimental.pallas import tpu as pltpu
from jax.experimental.pallas import tpu_sc as plsc

NC = 2
NS = 16
NW = NC * NS
D = 128
CPT = D // NW
NPAD = 10240
RPT = NPAD // NW
ECH = 8192
NG = 500

_MESH = plsc.VectorSubcoreMesh(
    core_axis_name="c", subcore_axis_name="s", num_cores=NC, num_subcores=NS)
_CP = pltpu.CompilerParams(needs_layout_passes=False)


def _wid():
  return lax.axis_index("c") * NS + lax.axis_index("s")


def _embed_body(z3, table, x0, idxv, rowsv, sem):
  w = _wid()
  pltpu.sync_copy(z3.at[w, 0], idxv)
  descs = []
  for i in range(RPT // 16):
    iv = idxv[pl.ds(i * 16, 16)]
    descs.append(pltpu.async_copy(table.at[iv],
                                  rowsv.at[pl.ds(i * 16, 16)], sem))
  for d in descs:
    d.wait()
  pltpu.sync_copy(rowsv, x0.at[pl.ds(w * RPT, RPT)])


def _embed(z3, table):
  return pl.kernel(
      _embed_body,
      out_type=jax.ShapeDtypeStruct((NPAD, D), jnp.float32),
      mesh=_MESH,
      compiler_params=_CP,
      scratch_types=[
          pltpu.VMEM((RPT,), jnp.int32),
          pltpu.VMEM((RPT, D), jnp.float32),
          pltpu.SemaphoreType.DMA,
      ],
  )(z3, table)


def _deg_body(tpe, dst3, ew3, degp, dstv, eww, acc):
  w = _wid()
  pltpu.sync_copy(dst3.at[w, 0], dstv)
  pltpu.sync_copy(ew3.at[w, 0], eww)
  z16 = jnp.zeros((16,), jnp.float32)

  def zf(i, _):
    acc[pl.ds(i * 16, 16)] = z16
    return 0
  lax.fori_loop(0, NPAD // 16, zf, 0)

  def body(g2, _):
    sl = pl.ds(g2 * 16, 16)
    plsc.addupdate_scatter(acc, [dstv[sl]], eww[sl])
    return 0
  lax.fori_loop(0, tpe // 16, body, 0)
  pltpu.sync_copy(acc, degp.at[w, 0])


def _deg(dst3, ew3):
  tpe = dst3.shape[2]
  return pl.kernel(
      lambda *a: _deg_body(tpe, *a),
      out_type=jax.ShapeDtypeStruct((NW, 1, NPAD), jnp.float32),
      mesh=_MESH,
      compiler_params=_CP,
      scratch_types=[
          pltpu.VMEM((tpe,), jnp.int32),
          pltpu.VMEM((tpe,), jnp.float32),
          pltpu.VMEM((NPAD,), jnp.float32),
      ],
  )(dst3, ew3)


def _agg_body(nch, gT, srcf, dstf, ewf, aggT, gcol, acc, srcv, dstv, eww):
  w = _wid()
  pltpu.sync_copy(gT.at[w, 0], gcol)
  z16 = jnp.zeros((16,), jnp.float32)

  def zf(i, _):
    acc[pl.ds(i * 16, 16)] = z16
    return 0
  lax.fori_loop(0, NPAD * CPT // 16, zf, 0)

  for k in range(nch):
    pltpu.sync_copy(srcf.at[0, 0, pl.ds(k * ECH, ECH)], srcv)
    pltpu.sync_copy(dstf.at[0, 0, pl.ds(k * ECH, ECH)], dstv)
    pltpu.sync_copy(ewf.at[0, 0, pl.ds(k * ECH, ECH)], eww)

    def body(g2, _):
      sl = pl.ds(g2 * 16, 16)
      sv = srcv[sl] * CPT
      dv = dstv[sl] * CPT
      ev = eww[sl]
      for c in range(CPT):
        vals = plsc.load_gather(gcol, [sv + c])
        plsc.addupdate_scatter(acc, [dv + c], vals * ev)
      return 0
    lax.fori_loop(0, ECH // 16, body, 0)
  pltpu.sync_copy(acc, aggT.at[w, 0])


def _agg(gT, srcf, dstf, ewf):
  nch = srcf.shape[2] // ECH
  return pl.kernel(
      lambda *a: _agg_body(nch, *a),
      out_type=jax.ShapeDtypeStruct((NW, 1, NPAD * CPT), jnp.float32),
      mesh=_MESH,
      compiler_params=_CP,
      scratch_types=[
          pltpu.VMEM((NPAD * CPT,), jnp.float32),
          pltpu.VMEM((NPAD * CPT,), jnp.float32),
          pltpu.VMEM((ECH,), jnp.int32),
          pltpu.VMEM((ECH,), jnp.int32),
          pltpu.VMEM((ECH,), jnp.float32),
      ],
  )(gT, srcf, dstf, ewf)


def _readout_body(p, dinvb, b3, c2, out, cidx, r0, rd, b3v, xbuf, prodv, sem):
  w = _wid()
  pltpu.sync_copy(c2.at[w, 0], cidx)
  descs = []
  for i in range(2):
    iv = cidx[pl.ds(i * 16, 16)]
    sl = pl.ds(i * 16, 16)
    descs.append(pltpu.async_copy(p.at[iv], r0.at[sl], sem))
    descs.append(pltpu.async_copy(dinvb.at[iv], rd.at[sl], sem))
  for d in descs:
    d.wait()
  pltpu.sync_copy(b3, b3v)

  def body(e, _):
    for i in range(8):
      sl = pl.ds(i * 16, 16)
      xbuf[e, sl] = r0[e, sl] * rd[e, sl] + b3v[sl]
    return 0
  lax.fori_loop(0, 32, body, 0)

  def body2(g2, _):
    for i in range(8):
      sl = pl.ds(i * 16, 16)
      prodv[g2, sl] = xbuf[2 * g2, sl] * xbuf[2 * g2 + 1, sl]
    return 0
  lax.fori_loop(0, 16, body2, 0)
  pltpu.sync_copy(prodv, out.at[pl.ds(w * 16, 16)])


def _readout(p, dinvb, b3, c2):
  return pl.kernel(
      _readout_body,
      out_type=jax.ShapeDtypeStruct((NW * 16, D), jnp.float32),
      mesh=_MESH,
      compiler_params=_CP,
      scratch_types=[
          pltpu.VMEM((32,), jnp.int32),
          pltpu.VMEM((32, D), jnp.float32),
          pltpu.VMEM((32, D), jnp.float32),
          pltpu.VMEM((D,), jnp.float32),
          pltpu.VMEM((32, D), jnp.float32),
          pltpu.VMEM((16, D), jnp.float32),
          pltpu.SemaphoreType.DMA,
      ],
  )(p, dinvb, b3, c2)


_R = 1280


def _conv1_tc(degT, x0, W1):
  def body(dt, x, wr, g_ref, dinv_ref):
    deg = jnp.sum(dt[...], axis=1, keepdims=True)
    dinv = jnp.where(deg > 0, lax.rsqrt(deg), 0.0)
    g_ref[...] = dinv * jnp.dot(x[...], wr[...],
                                preferred_element_type=jnp.float32)
    dinv_ref[...] = jnp.broadcast_to(dinv, (_R, D))

  return pl.pallas_call(
      body,
      grid=(NPAD // _R,),
      in_specs=[
          pl.BlockSpec((_R, NW), lambda i: (i, 0)),
          pl.BlockSpec((_R, D), lambda i: (i, 0)),
          pl.BlockSpec((D, D), lambda i: (0, 0)),
      ],
      out_specs=[
          pl.BlockSpec((_R, D), lambda i: (i, 0)),
          pl.BlockSpec((_R, D), lambda i: (i, 0)),
      ],
      out_shape=[
          jax.ShapeDtypeStruct((NPAD, D), jnp.float32),
          jax.ShapeDtypeStruct((NPAD, D), jnp.float32),
      ],
  )(degT, x0, W1)


def _conv_next_tc(agg, dinvb, bprev, W):
  def body(a0, dv, br, wr, g_ref):
    x = jax.nn.relu(dv[...] * a0[...] + br[...])
    g_ref[...] = dv[...] * jnp.dot(x, wr[...],
                                   preferred_element_type=jnp.float32)

  return pl.pallas_call(
      body,
      grid=(NPAD // _R,),
      in_specs=[
          pl.BlockSpec((_R, D), lambda i: (i, 0)),
          pl.BlockSpec((_R, D), lambda i: (i, 0)),
          pl.BlockSpec((1, D), lambda i: (0, 0)),
          pl.BlockSpec((D, D), lambda i: (0, 0)),
      ],
      out_specs=pl.BlockSpec((_R, D), lambda i: (i, 0)),
      out_shape=jax.ShapeDtypeStruct((NPAD, D), jnp.float32),
  )(agg, dinvb, bprev, W)


def _mlp_tc(prod, l1W, l1b, l2Wp, l2bp):
  def body(pr, w1r, b1r, w2r, b2r, o_ref):
    h = jax.nn.relu(jnp.dot(pr[...], w1r[...],
                            preferred_element_type=jnp.float32) + b1r[...])
    o_ref[...] = jnp.dot(h, w2r[...],
                         preferred_element_type=jnp.float32) + b2r[...]

  B = prod.shape[0]
  return pl.pallas_call(
      body,
      grid=(1,),
      in_specs=[
          pl.BlockSpec((B, D), lambda i: (0, 0)),
          pl.BlockSpec((D, D), lambda i: (0, 0)),
          pl.BlockSpec((1, D), lambda i: (0, 0)),
          pl.BlockSpec((D, D), lambda i: (0, 0)),
          pl.BlockSpec((1, D), lambda i: (0, 0)),
      ],
      out_specs=pl.BlockSpec((B, D), lambda i: (0, 0)),
      out_shape=jax.ShapeDtypeStruct((B, D), jnp.float32),
  )(prod, l1W, l1b, l2Wp, l2bp)


def _to_colblocks(g):
  return g.reshape(NPAD, NW, CPT).transpose(1, 0, 2).reshape(NW, 1,
                                                             NPAD * CPT)


def _from_colblocks(aggT):
  return aggT.reshape(NW, NPAD, CPT).transpose(1, 0, 2).reshape(NPAD, D)


def kernel(z, edge_index, batch, edge_weight, z_table, W1, b1, W2, b2, W3,
           b3, l1W, l1b, l2W, l2b):
  N = z.shape[0]
  loop = jnp.arange(N, dtype=jnp.int32)
  src = jnp.concatenate([edge_index[0].astype(jnp.int32), loop])
  dst = jnp.concatenate([edge_index[1].astype(jnp.int32), loop])
  ewf = jnp.concatenate([edge_weight.astype(jnp.float32),
                         jnp.ones((N,), jnp.float32)])
  EF = src.shape[0]
  EP = -(-EF // (NW * ECH)) * (NW * ECH)
  tpe = EP // NW
  srcp = jnp.pad(src, (0, EP - EF))
  dstp = jnp.pad(dst, (0, EP - EF))
  ewp = jnp.pad(ewf, (0, EP - EF))
  srcf = srcp.reshape(1, 1, EP)
  dstf = dstp.reshape(1, 1, EP)
  ewfl = ewp.reshape(1, 1, EP)
  dst3 = dstp.reshape(NW, 1, tpe)
  ew3 = ewp.reshape(NW, 1, tpe)
  z3 = jnp.pad(z.astype(jnp.int32), (0, NPAD - N)).reshape(NW, 1, RPT)

  center = jnp.searchsorted(batch.astype(jnp.int32),
                            jnp.arange(NG, dtype=jnp.int32)).astype(jnp.int32)
  c2 = jnp.stack([center, center + 1], axis=1).reshape(-1)
  c2 = jnp.pad(c2, (0, 1024 - 2 * NG)).reshape(NW, 1, 32)

  x0 = _embed(z3, z_table)
  degp = _deg(dst3, ew3)
  degT = degp.reshape(NW, NPAD).T
  g1, dinvb = _conv1_tc(degT, x0, W1)
  agg = _from_colblocks(_agg(_to_colblocks(g1), srcf, dstf, ewfl))
  g2 = _conv_next_tc(agg, dinvb, b1.reshape(1, D), W2)
  agg = _from_colblocks(_agg(_to_colblocks(g2), srcf, dstf, ewfl))
  g3 = _conv_next_tc(agg, dinvb, b2.reshape(1, D), W3)
  agg = _from_colblocks(_agg(_to_colblocks(g3), srcf, dstf, ewfl))
  prod = _readout(agg, dinvb, b3.astype(jnp.float32), c2)

  l2Wp = jnp.zeros((D, D), jnp.float32).at[:, 0].set(l2W[:, 0])
  l2bp = jnp.broadcast_to(l2b.astype(jnp.float32), (1, D))
  out = _mlp_tc(prod, l1W, l1b.reshape(1, D), l2Wp, l2bp)
  return out[:NG, :1]


# speedup vs baseline: 1.7666x; 1.0133x over previous
import jax
import jax.numpy as jnp
from jax import lax
from jax.experimental import pallas as pl
from jax.experimental.pallas import tpu as pltpu
from jax.experimental.pallas import tpu_sc as plsc

NC = 2
NS = 16
NW = NC * NS
D = 128
CPT = D // NW
NPAD = 10240
RPT = NPAD // NW
ECH = 8192
NG = 500

_MESH = plsc.VectorSubcoreMesh(
    core_axis_name="c", subcore_axis_name="s", num_cores=NC, num_subcores=NS)
_CP = pltpu.CompilerParams(needs_layout_passes=False)


def _wid():
  return lax.axis_index("c") * NS + lax.axis_index("s")


def _embed_body(z3, table, x0, idxv, rowsv, sem):
  w = _wid()
  pltpu.sync_copy(z3.at[w, 0], idxv)
  descs = []
  for i in range(RPT // 16):
    iv = idxv[pl.ds(i * 16, 16)]
    descs.append(pltpu.async_copy(table.at[iv],
                                  rowsv.at[pl.ds(i * 16, 16)], sem))
  for d in descs:
    d.wait()
  pltpu.sync_copy(rowsv, x0.at[pl.ds(w * RPT, RPT)])


def _embed(z3, table):
  return pl.kernel(
      _embed_body,
      out_type=jax.ShapeDtypeStruct((NPAD, D), jnp.float32),
      mesh=_MESH,
      compiler_params=_CP,
      scratch_types=[
          pltpu.VMEM((RPT,), jnp.int32),
          pltpu.VMEM((RPT, D), jnp.float32),
          pltpu.SemaphoreType.DMA,
      ],
  )(z3, table)


def _deg_body(tpe, dst3, ew3, degp, dstv, eww, acc):
  w = _wid()
  pltpu.sync_copy(dst3.at[w, 0], dstv)
  pltpu.sync_copy(ew3.at[w, 0], eww)
  z16 = jnp.zeros((16,), jnp.float32)

  def zf(i, _):
    acc[pl.ds(i * 16, 16)] = z16
    return 0
  lax.fori_loop(0, NPAD // 16, zf, 0)

  def body(g2, _):
    sl = pl.ds(g2 * 16, 16)
    plsc.addupdate_scatter(acc, [dstv[sl]], eww[sl])
    return 0
  lax.fori_loop(0, tpe // 16, body, 0, unroll=4)
  pltpu.sync_copy(acc, degp.at[w, 0])


def _deg(dst3, ew3):
  tpe = dst3.shape[2]
  return pl.kernel(
      lambda *a: _deg_body(tpe, *a),
      out_type=jax.ShapeDtypeStruct((NW, 1, NPAD), jnp.float32),
      mesh=_MESH,
      compiler_params=_CP,
      scratch_types=[
          pltpu.VMEM((tpe,), jnp.int32),
          pltpu.VMEM((tpe,), jnp.float32),
          pltpu.VMEM((NPAD,), jnp.float32),
      ],
  )(dst3, ew3)


def _agg_body(nch, gT, srcf, dstf, ewf, aggT, gcol, acc, srcv, dstv, eww):
  w = _wid()
  pltpu.sync_copy(gT.at[w, 0], gcol)
  z16 = jnp.zeros((16,), jnp.float32)

  def zf(i, _):
    acc[pl.ds(i * 16, 16)] = z16
    return 0
  lax.fori_loop(0, NPAD * CPT // 16, zf, 0, unroll=8)

  for k in range(nch):
    pltpu.sync_copy(srcf.at[0, 0, pl.ds(k * ECH, ECH)], srcv)
    pltpu.sync_copy(dstf.at[0, 0, pl.ds(k * ECH, ECH)], dstv)
    pltpu.sync_copy(ewf.at[0, 0, pl.ds(k * ECH, ECH)], eww)

    def body(g2, _):
      sl = pl.ds(g2 * 16, 16)
      sv = srcv[sl] * CPT
      dv = dstv[sl] * CPT
      ev = eww[sl]
      for c in range(CPT):
        vals = plsc.load_gather(gcol, [sv + c])
        plsc.addupdate_scatter(acc, [dv + c], vals * ev)
      return 0
    lax.fori_loop(0, ECH // 16, body, 0, unroll=4)
  pltpu.sync_copy(acc, aggT.at[w, 0])


def _agg(gT, srcf, dstf, ewf):
  nch = srcf.shape[2] // ECH
  return pl.kernel(
      lambda *a: _agg_body(nch, *a),
      out_type=jax.ShapeDtypeStruct((NW, 1, NPAD * CPT), jnp.float32),
      mesh=_MESH,
      compiler_params=_CP,
      scratch_types=[
          pltpu.VMEM((NPAD * CPT,), jnp.float32),
          pltpu.VMEM((NPAD * CPT,), jnp.float32),
          pltpu.VMEM((ECH,), jnp.int32),
          pltpu.VMEM((ECH,), jnp.int32),
          pltpu.VMEM((ECH,), jnp.float32),
      ],
  )(gT, srcf, dstf, ewf)


def _readout_body(p, dinvb, b3, c2, out, cidx, r0, rd, b3v, xbuf, prodv, sem):
  w = _wid()
  pltpu.sync_copy(c2.at[w, 0], cidx)
  descs = []
  for i in range(2):
    iv = cidx[pl.ds(i * 16, 16)]
    sl = pl.ds(i * 16, 16)
    descs.append(pltpu.async_copy(p.at[iv], r0.at[sl], sem))
    descs.append(pltpu.async_copy(dinvb.at[iv], rd.at[sl], sem))
  for d in descs:
    d.wait()
  pltpu.sync_copy(b3, b3v)

  def body(e, _):
    for i in range(8):
      sl = pl.ds(i * 16, 16)
      xbuf[e, sl] = r0[e, sl] * rd[e, sl] + b3v[sl]
    return 0
  lax.fori_loop(0, 32, body, 0)

  def body2(g2, _):
    for i in range(8):
      sl = pl.ds(i * 16, 16)
      prodv[g2, sl] = xbuf[2 * g2, sl] * xbuf[2 * g2 + 1, sl]
    return 0
  lax.fori_loop(0, 16, body2, 0)
  pltpu.sync_copy(prodv, out.at[pl.ds(w * 16, 16)])


def _readout(p, dinvb, b3, c2):
  return pl.kernel(
      _readout_body,
      out_type=jax.ShapeDtypeStruct((NW * 16, D), jnp.float32),
      mesh=_MESH,
      compiler_params=_CP,
      scratch_types=[
          pltpu.VMEM((32,), jnp.int32),
          pltpu.VMEM((32, D), jnp.float32),
          pltpu.VMEM((32, D), jnp.float32),
          pltpu.VMEM((D,), jnp.float32),
          pltpu.VMEM((32, D), jnp.float32),
          pltpu.VMEM((16, D), jnp.float32),
          pltpu.SemaphoreType.DMA,
      ],
  )(p, dinvb, b3, c2)


_R = 1280


def _conv1_tc(degT, x0, W1):
  def body(dt, x, wr, g_ref, dinv_ref):
    deg = jnp.sum(dt[...], axis=1, keepdims=True)
    dinv = jnp.where(deg > 0, lax.rsqrt(deg), 0.0)
    g_ref[...] = dinv * jnp.dot(x[...], wr[...],
                                preferred_element_type=jnp.float32)
    dinv_ref[...] = jnp.broadcast_to(dinv, (_R, D))

  return pl.pallas_call(
      body,
      grid=(NPAD // _R,),
      in_specs=[
          pl.BlockSpec((_R, NW), lambda i: (i, 0)),
          pl.BlockSpec((_R, D), lambda i: (i, 0)),
          pl.BlockSpec((D, D), lambda i: (0, 0)),
      ],
      out_specs=[
          pl.BlockSpec((_R, D), lambda i: (i, 0)),
          pl.BlockSpec((_R, D), lambda i: (i, 0)),
      ],
      out_shape=[
          jax.ShapeDtypeStruct((NPAD, D), jnp.float32),
          jax.ShapeDtypeStruct((NPAD, D), jnp.float32),
      ],
  )(degT, x0, W1)


def _conv_next_tc(agg, dinvb, bprev, W):
  def body(a0, dv, br, wr, g_ref):
    x = jax.nn.relu(dv[...] * a0[...] + br[...])
    g_ref[...] = dv[...] * jnp.dot(x, wr[...],
                                   preferred_element_type=jnp.float32)

  return pl.pallas_call(
      body,
      grid=(NPAD // _R,),
      in_specs=[
          pl.BlockSpec((_R, D), lambda i: (i, 0)),
          pl.BlockSpec((_R, D), lambda i: (i, 0)),
          pl.BlockSpec((1, D), lambda i: (0, 0)),
          pl.BlockSpec((D, D), lambda i: (0, 0)),
      ],
      out_specs=pl.BlockSpec((_R, D), lambda i: (i, 0)),
      out_shape=jax.ShapeDtypeStruct((NPAD, D), jnp.float32),
  )(agg, dinvb, bprev, W)


def _mlp_tc(prod, l1W, l1b, l2Wp, l2bp):
  def body(pr, w1r, b1r, w2r, b2r, o_ref):
    h = jax.nn.relu(jnp.dot(pr[...], w1r[...],
                            preferred_element_type=jnp.float32) + b1r[...])
    o_ref[...] = jnp.dot(h, w2r[...],
                         preferred_element_type=jnp.float32) + b2r[...]

  B = prod.shape[0]
  return pl.pallas_call(
      body,
      grid=(1,),
      in_specs=[
          pl.BlockSpec((B, D), lambda i: (0, 0)),
          pl.BlockSpec((D, D), lambda i: (0, 0)),
          pl.BlockSpec((1, D), lambda i: (0, 0)),
          pl.BlockSpec((D, D), lambda i: (0, 0)),
          pl.BlockSpec((1, D), lambda i: (0, 0)),
      ],
      out_specs=pl.BlockSpec((B, D), lambda i: (0, 0)),
      out_shape=jax.ShapeDtypeStruct((B, D), jnp.float32),
  )(prod, l1W, l1b, l2Wp, l2bp)


def _to_colblocks(g):
  return g.reshape(NPAD, NW, CPT).transpose(1, 0, 2).reshape(NW, 1,
                                                             NPAD * CPT)


def _from_colblocks(aggT):
  return aggT.reshape(NW, NPAD, CPT).transpose(1, 0, 2).reshape(NPAD, D)


def kernel(z, edge_index, batch, edge_weight, z_table, W1, b1, W2, b2, W3,
           b3, l1W, l1b, l2W, l2b):
  N = z.shape[0]
  loop = jnp.arange(N, dtype=jnp.int32)
  src = jnp.concatenate([edge_index[0].astype(jnp.int32), loop])
  dst = jnp.concatenate([edge_index[1].astype(jnp.int32), loop])
  ewf = jnp.concatenate([edge_weight.astype(jnp.float32),
                         jnp.ones((N,), jnp.float32)])
  EF = src.shape[0]
  EP = -(-EF // (NW * ECH)) * (NW * ECH)
  tpe = EP // NW
  srcp = jnp.pad(src, (0, EP - EF))
  dstp = jnp.pad(dst, (0, EP - EF))
  ewp = jnp.pad(ewf, (0, EP - EF))
  srcf = srcp.reshape(1, 1, EP)
  dstf = dstp.reshape(1, 1, EP)
  ewfl = ewp.reshape(1, 1, EP)
  dst3 = dstp.reshape(NW, 1, tpe)
  ew3 = ewp.reshape(NW, 1, tpe)
  z3 = jnp.pad(z.astype(jnp.int32), (0, NPAD - N)).reshape(NW, 1, RPT)

  center = jnp.searchsorted(batch.astype(jnp.int32),
                            jnp.arange(NG, dtype=jnp.int32)).astype(jnp.int32)
  c2 = jnp.stack([center, center + 1], axis=1).reshape(-1)
  c2 = jnp.pad(c2, (0, 1024 - 2 * NG)).reshape(NW, 1, 32)

  x0 = _embed(z3, z_table)
  degp = _deg(dst3, ew3)
  degT = degp.reshape(NW, NPAD).T
  g1, dinvb = _conv1_tc(degT, x0, W1)
  agg = _from_colblocks(_agg(_to_colblocks(g1), srcf, dstf, ewfl))
  g2 = _conv_next_tc(agg, dinvb, b1.reshape(1, D), W2)
  agg = _from_colblocks(_agg(_to_colblocks(g2), srcf, dstf, ewfl))
  g3 = _conv_next_tc(agg, dinvb, b2.reshape(1, D), W3)
  agg = _from_colblocks(_agg(_to_colblocks(g3), srcf, dstf, ewfl))
  prod = _readout(agg, dinvb, b3.astype(jnp.float32), c2)

  l2Wp = jnp.zeros((D, D), jnp.float32).at[:, 0].set(l2W[:, 0])
  l2bp = jnp.broadcast_to(l2b.astype(jnp.float32), (1, D))
  out = _mlp_tc(prod, l1W, l1b.reshape(1, D), l2Wp, l2bp)
  return out[:NG, :1]


# speedup vs baseline: 2.6595x; 1.5054x over previous
import jax
import jax.numpy as jnp
from jax import lax
from jax.experimental import pallas as pl
from jax.experimental.pallas import tpu as pltpu
from jax.experimental.pallas import tpu_sc as plsc

NC = 2
NS = 16
NW = NC * NS
D = 128
CPT = D // NW
NPAD = 10240
RPT = NPAD // NW
ECH = 8192
NG = 500

_MESH = plsc.VectorSubcoreMesh(
    core_axis_name="c", subcore_axis_name="s", num_cores=NC, num_subcores=NS)
_CP = pltpu.CompilerParams(needs_layout_passes=False)


def _wid():
  return lax.axis_index("c") * NS + lax.axis_index("s")


def _embed_body(z3, table, x0, idxv, rowsv, sem):
  w = _wid()
  pltpu.sync_copy(z3.at[w, 0], idxv)
  descs = []
  for i in range(RPT // 16):
    iv = idxv[pl.ds(i * 16, 16)]
    descs.append(pltpu.async_copy(table.at[iv],
                                  rowsv.at[pl.ds(i * 16, 16)], sem))
  for d in descs:
    d.wait()
  pltpu.sync_copy(rowsv, x0.at[pl.ds(w * RPT, RPT)])


def _embed(z3, table):
  return pl.kernel(
      _embed_body,
      out_type=jax.ShapeDtypeStruct((NPAD, D), jnp.float32),
      mesh=_MESH,
      compiler_params=_CP,
      scratch_types=[
          pltpu.VMEM((RPT,), jnp.int32),
          pltpu.VMEM((RPT, D), jnp.float32),
          pltpu.SemaphoreType.DMA,
      ],
  )(z3, table)


def _deg_body(tpe, dst3, ew3, degp, dstv, eww, acc):
  w = _wid()
  pltpu.sync_copy(dst3.at[w, 0], dstv)
  pltpu.sync_copy(ew3.at[w, 0], eww)
  z16 = jnp.zeros((16,), jnp.float32)

  def zf(i, _):
    acc[pl.ds(i * 16, 16)] = z16
    return 0
  lax.fori_loop(0, NPAD // 16, zf, 0)

  def body(g2, _):
    sl = pl.ds(g2 * 16, 16)
    plsc.addupdate_scatter(acc, [dstv[sl]], eww[sl])
    return 0
  lax.fori_loop(0, tpe // 16, body, 0, unroll=4)
  pltpu.sync_copy(acc, degp.at[w, 0])


def _deg(dst3, ew3):
  tpe = dst3.shape[2]
  return pl.kernel(
      lambda *a: _deg_body(tpe, *a),
      out_type=jax.ShapeDtypeStruct((NW, 1, NPAD), jnp.float32),
      mesh=_MESH,
      compiler_params=_CP,
      scratch_types=[
          pltpu.VMEM((tpe,), jnp.int32),
          pltpu.VMEM((tpe,), jnp.float32),
          pltpu.VMEM((NPAD,), jnp.float32),
      ],
  )(dst3, ew3)


def _agg_body(nch, gT, srcf, dstf, ewf, aggT, gcol, acc, srcv, dstv, eww):
  w = _wid()
  pltpu.sync_copy(gT.at[w, 0], gcol)
  z16 = jnp.zeros((16,), jnp.float32)

  def zf(i, _):
    acc[pl.ds(i * 16, 16)] = z16
    return 0
  lax.fori_loop(0, NPAD * CPT // 16, zf, 0, unroll=8)

  for k in range(nch):
    pltpu.sync_copy(srcf.at[0, 0, pl.ds(k * ECH, ECH)], srcv)
    pltpu.sync_copy(dstf.at[0, 0, pl.ds(k * ECH, ECH)], dstv)
    pltpu.sync_copy(ewf.at[0, 0, pl.ds(k * ECH, ECH)], eww)

    def body(g2, _):
      svs, dvs, evs = [], [], []
      for a in range(4):
        sl = pl.ds(g2 * 64 + a * 16, 16)
        svs.append(srcv[sl] * CPT)
        dvs.append(dstv[sl] * CPT)
        evs.append(eww[sl])
      vals = []
      for a in range(4):
        for c in range(CPT):
          vals.append(plsc.load_gather(gcol, [svs[a] + c]) * evs[a])
      for a in range(4):
        for c in range(CPT):
          plsc.addupdate_scatter(acc, [dvs[a] + c], vals[a * CPT + c])
      return 0
    lax.fori_loop(0, ECH // 64, body, 0)
  pltpu.sync_copy(acc, aggT.at[w, 0])


def _agg(gT, srcf, dstf, ewf):
  nch = srcf.shape[2] // ECH
  return pl.kernel(
      lambda *a: _agg_body(nch, *a),
      out_type=jax.ShapeDtypeStruct((NW, 1, NPAD * CPT), jnp.float32),
      mesh=_MESH,
      compiler_params=_CP,
      scratch_types=[
          pltpu.VMEM((NPAD * CPT,), jnp.float32),
          pltpu.VMEM((NPAD * CPT,), jnp.float32),
          pltpu.VMEM((ECH,), jnp.int32),
          pltpu.VMEM((ECH,), jnp.int32),
          pltpu.VMEM((ECH,), jnp.float32),
      ],
  )(gT, srcf, dstf, ewf)


def _readout_body(p, dinvb, b3, c2, out, cidx, r0, rd, b3v, xbuf, prodv, sem):
  w = _wid()
  pltpu.sync_copy(c2.at[w, 0], cidx)
  descs = []
  for i in range(2):
    iv = cidx[pl.ds(i * 16, 16)]
    sl = pl.ds(i * 16, 16)
    descs.append(pltpu.async_copy(p.at[iv], r0.at[sl], sem))
    descs.append(pltpu.async_copy(dinvb.at[iv], rd.at[sl], sem))
  for d in descs:
    d.wait()
  pltpu.sync_copy(b3, b3v)

  def body(e, _):
    for i in range(8):
      sl = pl.ds(i * 16, 16)
      xbuf[e, sl] = r0[e, sl] * rd[e, sl] + b3v[sl]
    return 0
  lax.fori_loop(0, 32, body, 0)

  def body2(g2, _):
    for i in range(8):
      sl = pl.ds(i * 16, 16)
      prodv[g2, sl] = xbuf[2 * g2, sl] * xbuf[2 * g2 + 1, sl]
    return 0
  lax.fori_loop(0, 16, body2, 0)
  pltpu.sync_copy(prodv, out.at[pl.ds(w * 16, 16)])


def _readout(p, dinvb, b3, c2):
  return pl.kernel(
      _readout_body,
      out_type=jax.ShapeDtypeStruct((NW * 16, D), jnp.float32),
      mesh=_MESH,
      compiler_params=_CP,
      scratch_types=[
          pltpu.VMEM((32,), jnp.int32),
          pltpu.VMEM((32, D), jnp.float32),
          pltpu.VMEM((32, D), jnp.float32),
          pltpu.VMEM((D,), jnp.float32),
          pltpu.VMEM((32, D), jnp.float32),
          pltpu.VMEM((16, D), jnp.float32),
          pltpu.SemaphoreType.DMA,
      ],
  )(p, dinvb, b3, c2)


_R = 1280


def _conv1_tc(degT, x0, W1):
  def body(dt, x, wr, g_ref, dinv_ref):
    deg = jnp.sum(dt[...], axis=1, keepdims=True)
    dinv = jnp.where(deg > 0, lax.rsqrt(deg), 0.0)
    g_ref[...] = dinv * jnp.dot(x[...], wr[...],
                                preferred_element_type=jnp.float32)
    dinv_ref[...] = jnp.broadcast_to(dinv, (_R, D))

  return pl.pallas_call(
      body,
      grid=(NPAD // _R,),
      in_specs=[
          pl.BlockSpec((_R, NW), lambda i: (i, 0)),
          pl.BlockSpec((_R, D), lambda i: (i, 0)),
          pl.BlockSpec((D, D), lambda i: (0, 0)),
      ],
      out_specs=[
          pl.BlockSpec((_R, D), lambda i: (i, 0)),
          pl.BlockSpec((_R, D), lambda i: (i, 0)),
      ],
      out_shape=[
          jax.ShapeDtypeStruct((NPAD, D), jnp.float32),
          jax.ShapeDtypeStruct((NPAD, D), jnp.float32),
      ],
  )(degT, x0, W1)


def _conv_next_tc(agg, dinvb, bprev, W):
  def body(a0, dv, br, wr, g_ref):
    x = jax.nn.relu(dv[...] * a0[...] + br[...])
    g_ref[...] = dv[...] * jnp.dot(x, wr[...],
                                   preferred_element_type=jnp.float32)

  return pl.pallas_call(
      body,
      grid=(NPAD // _R,),
      in_specs=[
          pl.BlockSpec((_R, D), lambda i: (i, 0)),
          pl.BlockSpec((_R, D), lambda i: (i, 0)),
          pl.BlockSpec((1, D), lambda i: (0, 0)),
          pl.BlockSpec((D, D), lambda i: (0, 0)),
      ],
      out_specs=pl.BlockSpec((_R, D), lambda i: (i, 0)),
      out_shape=jax.ShapeDtypeStruct((NPAD, D), jnp.float32),
  )(agg, dinvb, bprev, W)


def _mlp_tc(prod, l1W, l1b, l2Wp, l2bp):
  def body(pr, w1r, b1r, w2r, b2r, o_ref):
    h = jax.nn.relu(jnp.dot(pr[...], w1r[...],
                            preferred_element_type=jnp.float32) + b1r[...])
    o_ref[...] = jnp.dot(h, w2r[...],
                         preferred_element_type=jnp.float32) + b2r[...]

  B = prod.shape[0]
  return pl.pallas_call(
      body,
      grid=(1,),
      in_specs=[
          pl.BlockSpec((B, D), lambda i: (0, 0)),
          pl.BlockSpec((D, D), lambda i: (0, 0)),
          pl.BlockSpec((1, D), lambda i: (0, 0)),
          pl.BlockSpec((D, D), lambda i: (0, 0)),
          pl.BlockSpec((1, D), lambda i: (0, 0)),
      ],
      out_specs=pl.BlockSpec((B, D), lambda i: (0, 0)),
      out_shape=jax.ShapeDtypeStruct((B, D), jnp.float32),
  )(prod, l1W, l1b, l2Wp, l2bp)


def _to_colblocks(g):
  return g.reshape(NPAD, NW, CPT).transpose(1, 0, 2).reshape(NW, 1,
                                                             NPAD * CPT)


def _from_colblocks(aggT):
  return aggT.reshape(NW, NPAD, CPT).transpose(1, 0, 2).reshape(NPAD, D)


def kernel(z, edge_index, batch, edge_weight, z_table, W1, b1, W2, b2, W3,
           b3, l1W, l1b, l2W, l2b):
  N = z.shape[0]
  loop = jnp.arange(N, dtype=jnp.int32)
  src = jnp.concatenate([edge_index[0].astype(jnp.int32), loop])
  dst = jnp.concatenate([edge_index[1].astype(jnp.int32), loop])
  ewf = jnp.concatenate([edge_weight.astype(jnp.float32),
                         jnp.ones((N,), jnp.float32)])
  EF = src.shape[0]
  EP = -(-EF // (NW * ECH)) * (NW * ECH)
  tpe = EP // NW
  srcp = jnp.pad(src, (0, EP - EF))
  dstp = jnp.pad(dst, (0, EP - EF))
  ewp = jnp.pad(ewf, (0, EP - EF))
  srcf = srcp.reshape(1, 1, EP)
  dstf = dstp.reshape(1, 1, EP)
  ewfl = ewp.reshape(1, 1, EP)
  dst3 = dstp.reshape(NW, 1, tpe)
  ew3 = ewp.reshape(NW, 1, tpe)
  z3 = jnp.pad(z.astype(jnp.int32), (0, NPAD - N)).reshape(NW, 1, RPT)

  center = jnp.searchsorted(batch.astype(jnp.int32),
                            jnp.arange(NG, dtype=jnp.int32)).astype(jnp.int32)
  c2 = jnp.stack([center, center + 1], axis=1).reshape(-1)
  c2 = jnp.pad(c2, (0, 1024 - 2 * NG)).reshape(NW, 1, 32)

  x0 = _embed(z3, z_table)
  degp = _deg(dst3, ew3)
  degT = degp.reshape(NW, NPAD).T
  g1, dinvb = _conv1_tc(degT, x0, W1)
  agg = _from_colblocks(_agg(_to_colblocks(g1), srcf, dstf, ewfl))
  g2 = _conv_next_tc(agg, dinvb, b1.reshape(1, D), W2)
  agg = _from_colblocks(_agg(_to_colblocks(g2), srcf, dstf, ewfl))
  g3 = _conv_next_tc(agg, dinvb, b2.reshape(1, D), W3)
  agg = _from_colblocks(_agg(_to_colblocks(g3), srcf, dstf, ewfl))
  prod = _readout(agg, dinvb, b3.astype(jnp.float32), c2)

  l2Wp = jnp.zeros((D, D), jnp.float32).at[:, 0].set(l2W[:, 0])
  l2bp = jnp.broadcast_to(l2b.astype(jnp.float32), (1, D))
  out = _mlp_tc(prod, l1W, l1b.reshape(1, D), l2Wp, l2bp)
  return out[:NG, :1]
